# baseline (speedup 1.0000x reference)
import jax
import jax.numpy as jnp
from jax.experimental import pallas as pl
from jax.experimental.pallas import tpu as pltpu

_EPS = 1e-5
_VMEM_LIMIT = 48 * 1024 * 1024
_CP = getattr(pltpu, "CompilerParams", None) or getattr(
    pltpu, "TPUCompilerParams")


def _cparams(sem):
    return _CP(dimension_semantics=sem, vmem_limit_bytes=_VMEM_LIMIT)


def _im2col_conv(x, w_ref, cols_ref):
    H, W, Cin = x.shape
    Cout = w_ref.shape[-1]
    xb = x.astype(jnp.bfloat16)

    cols_ref[0:1] = jnp.zeros((1, W, 3 * Cin), jnp.bfloat16)
    cols_ref[H + 1:H + 2] = jnp.zeros((1, W, 3 * Cin), jnp.bfloat16)
    cols_ref[:, 0:1, 0:Cin] = jnp.zeros((H + 2, 1, Cin), jnp.bfloat16)
    cols_ref[:, W - 1:W, 2 * Cin:] = jnp.zeros((H + 2, 1, Cin), jnp.bfloat16)

    cols_ref[1:H + 1, 1:W, 0:Cin] = xb[:, 0:W - 1, :]
    cols_ref[1:H + 1, :, Cin:2 * Cin] = xb
    cols_ref[1:H + 1, 0:W - 1, 2 * Cin:] = xb[:, 1:W, :]

    acc = jnp.zeros((H * W, Cout), jnp.float32)
    for kh in range(3):
        acc = acc + jnp.dot(
            cols_ref[kh:kh + H].reshape(H * W, 3 * Cin),
            w_ref[kh],
            preferred_element_type=jnp.float32)
    return acc


def _conv1_kernel(x_ref, w_ref, o_ref, st_ref, cols_ref):
    _, H, W, Cout = o_ref.shape
    acc = _im2col_conv(x_ref[0], w_ref, cols_ref)
    o_ref[0] = acc.reshape(H, W, Cout).astype(jnp.bfloat16)
    st_ref[0, 0:1, :] = jnp.sum(acc, axis=0, keepdims=True)
    st_ref[0, 1:2, :] = jnp.sum(acc * acc, axis=0, keepdims=True)


def _conv2_kernel(x_ref, sc_ref, sh_ref, w_ref, o_ref, st_ref, cols_ref):
    _, H, W, Cout = o_ref.shape
    x = x_ref[0].astype(jnp.float32)
    y = jnp.maximum(x * sc_ref[...].reshape(1, 1, -1)
                    + sh_ref[...].reshape(1, 1, -1), 0.0)
    acc = _im2col_conv(y, w_ref, cols_ref)
    o_ref[0] = acc.reshape(H, W, Cout).astype(jnp.bfloat16)
    st_ref[0, 0:1, :] = jnp.sum(acc, axis=0, keepdims=True)
    st_ref[0, 1:2, :] = jnp.sum(acc * acc, axis=0, keepdims=True)


def _tail_kernel(x_ref, r_ref, sc_ref, sh_ref, o_ref):
    x = x_ref[0].astype(jnp.float32)
    r = r_ref[0].astype(jnp.float32)
    sc = sc_ref[...].reshape(1, 1, -1)
    sh = sh_ref[...].reshape(1, 1, -1)
    o_ref[0] = jnp.maximum(x * sc + sh + r, 0.0)


def _conv_stats(x_bf, w_bf, extra_in=None):
    N, H, W, Cin = x_bf.shape
    Cout = w_bf.shape[-1]
    if extra_in is None:
        kern = _conv1_kernel
        ins = [x_bf, w_bf]
        in_specs = [
            pl.BlockSpec((1, H, W, Cin), lambda n: (n, 0, 0, 0)),
            pl.BlockSpec(w_bf.shape, lambda n: (0, 0, 0)),
        ]
    else:
        scale, shift = extra_in
        kern = _conv2_kernel
        ins = [x_bf, scale, shift, w_bf]
        in_specs = [
            pl.BlockSpec((1, H, W, Cin), lambda n: (n, 0, 0, 0)),
            pl.BlockSpec((1, Cin), lambda n: (0, 0)),
            pl.BlockSpec((1, Cin), lambda n: (0, 0)),
            pl.BlockSpec(w_bf.shape, lambda n: (0, 0, 0)),
        ]
    return pl.pallas_call(
        kern,
        grid=(N,),
        in_specs=in_specs,
        out_specs=(
            pl.BlockSpec((1, H, W, Cout), lambda n: (n, 0, 0, 0)),
            pl.BlockSpec((1, 2, Cout), lambda n: (n, 0, 0)),
        ),
        out_shape=(
            jax.ShapeDtypeStruct((N, H, W, Cout), jnp.bfloat16),
            jax.ShapeDtypeStruct((N, 2, Cout), jnp.float32),
        ),
        scratch_shapes=[pltpu.VMEM((H + 2, W, 3 * Cin), jnp.bfloat16)],
        compiler_params=_cparams(("parallel",)),
    )(*ins)


def _fold(stats, count, gamma, beta):
    s = jnp.sum(stats[:, 0, :], axis=0)
    ss = jnp.sum(stats[:, 1, :], axis=0)
    mean = s / count
    var = ss / count - mean * mean
    scale = gamma * jax.lax.rsqrt(var + _EPS)
    shift = beta - mean * scale
    return scale.reshape(1, -1), shift.reshape(1, -1)


def kernel(x_nchw, w1, w2, g1, b1, g2, b2):
    x = jnp.transpose(x_nchw, (0, 2, 3, 1))
    x_bf = x.astype(jnp.bfloat16)
    N, H, W, C = x.shape
    count = N * H * W

    w1p = w1.reshape(3, 3 * C, C).astype(jnp.bfloat16)
    w2p = w2.reshape(3, 3 * C, C).astype(jnp.bfloat16)

    c1, st1 = _conv_stats(x_bf, w1p)
    sc1, sh1 = _fold(st1, count, g1, b1)
    c2, st2 = _conv_stats(c1, w2p, (sc1, sh1))
    sc2, sh2 = _fold(st2, count, g2, b2)

    th = 8
    out = pl.pallas_call(
        _tail_kernel,
        grid=(N, H // th),
        in_specs=[
            pl.BlockSpec((1, th, W, C), lambda n, h: (n, h, 0, 0)),
            pl.BlockSpec((1, th, W, C), lambda n, h: (n, h, 0, 0)),
            pl.BlockSpec((1, C), lambda n, h: (0, 0)),
            pl.BlockSpec((1, C), lambda n, h: (0, 0)),
        ],
        out_specs=pl.BlockSpec((1, th, W, C), lambda n, h: (n, h, 0, 0)),
        out_shape=jax.ShapeDtypeStruct((N, H, W, C), jnp.float32),
        compiler_params=_cparams(("parallel", "parallel")),
    )(c2, x_bf, sc2, sh2)
    return jnp.transpose(out, (0, 3, 1, 2))


# speedup vs baseline: 1.2912x; 1.2912x over previous
import functools

import jax
import jax.numpy as jnp
from jax.experimental import pallas as pl
from jax.experimental.pallas import tpu as pltpu

_EPS = 1e-5
_VMEM_LIMIT = 64 * 1024 * 1024
_CP = getattr(pltpu, "CompilerParams", None) or getattr(
    pltpu, "TPUCompilerParams")


def _cparams(sem):
    return _CP(dimension_semantics=sem, vmem_limit_bytes=_VMEM_LIMIT)


def _build_cols(xb, W, cols_ref):
    C, HW = xb.shape
    col = jax.lax.broadcasted_iota(jnp.int32, (1, HW), 1) % W
    has_l = col != 0
    has_r = col != (W - 1)
    zrow = jnp.zeros((C, W), jnp.bfloat16)
    z1 = jnp.zeros((C, 1), jnp.bfloat16)
    shifted = (
        jnp.concatenate([zrow, xb[:, :HW - W]], axis=1),
        xb,
        jnp.concatenate([xb[:, W:], zrow], axis=1),
    )
    zb = jnp.zeros((C, HW), jnp.bfloat16)
    for kh in range(3):
        r = shifted[kh]
        left = jnp.where(has_l, jnp.concatenate([z1, r[:, :HW - 1]], axis=1), zb)
        right = jnp.where(has_r, jnp.concatenate([r[:, 1:], z1], axis=1), zb)
        cols_ref[(kh * 3 + 0) * C:(kh * 3 + 1) * C] = left
        cols_ref[(kh * 3 + 1) * C:(kh * 3 + 2) * C] = r
        cols_ref[(kh * 3 + 2) * C:(kh * 3 + 3) * C] = right


def _stats(st_ref, acc):
    st_ref[0, :, 0:1] = jnp.sum(acc, axis=1, keepdims=True)
    st_ref[0, :, 1:2] = jnp.sum(acc * acc, axis=1, keepdims=True)


def _conv1_kernel(W, x_ref, w_ref, o_ref, st_ref, cols_ref):
    xb = x_ref[0].astype(jnp.bfloat16)
    _build_cols(xb, W, cols_ref)
    acc = jnp.dot(w_ref[...], cols_ref[...],
                  preferred_element_type=jnp.float32)
    o_ref[0] = acc.astype(jnp.bfloat16)
    _stats(st_ref, acc)


def _conv2_kernel(W, x_ref, sc_ref, sh_ref, w_ref, o_ref, st_ref, cols_ref):
    y = x_ref[0].astype(jnp.float32) * sc_ref[...] + sh_ref[...]
    yb = jnp.maximum(y, 0.0).astype(jnp.bfloat16)
    _build_cols(yb, W, cols_ref)
    acc = jnp.dot(w_ref[...], cols_ref[...],
                  preferred_element_type=jnp.float32)
    o_ref[0] = acc.astype(jnp.bfloat16)
    _stats(st_ref, acc)


def _tail_kernel(x_ref, r_ref, sc_ref, sh_ref, o_ref):
    x = x_ref[0].astype(jnp.float32)
    o_ref[0] = jnp.maximum(x * sc_ref[...] + sh_ref[...] + r_ref[0], 0.0)


def _conv_bnstats(x, w_t, W, scale=None, shift=None):
    N, C, HW = x.shape
    KC = w_t.shape[1]
    if scale is None:
        kern = functools.partial(_conv1_kernel, W)
        ins = [x, w_t]
        in_specs = [
            pl.BlockSpec((1, C, HW), lambda n: (n, 0, 0)),
            pl.BlockSpec((C, KC), lambda n: (0, 0)),
        ]
    else:
        kern = functools.partial(_conv2_kernel, W)
        ins = [x, scale, shift, w_t]
        in_specs = [
            pl.BlockSpec((1, C, HW), lambda n: (n, 0, 0)),
            pl.BlockSpec((C, 1), lambda n: (0, 0)),
            pl.BlockSpec((C, 1), lambda n: (0, 0)),
            pl.BlockSpec((C, KC), lambda n: (0, 0)),
        ]
    return pl.pallas_call(
        kern,
        grid=(N,),
        in_specs=in_specs,
        out_specs=(
            pl.BlockSpec((1, C, HW), lambda n: (n, 0, 0)),
            pl.BlockSpec((1, C, 2), lambda n: (n, 0, 0)),
        ),
        out_shape=(
            jax.ShapeDtypeStruct((N, C, HW), jnp.bfloat16),
            jax.ShapeDtypeStruct((N, C, 2), jnp.float32),
        ),
        scratch_shapes=[pltpu.VMEM((KC, HW), jnp.bfloat16)],
        compiler_params=_cparams(("parallel",)),
    )(*ins)


def _fold(st, count, gamma, beta):
    s = jnp.sum(st[:, :, 0], axis=0)
    ss = jnp.sum(st[:, :, 1], axis=0)
    mean = s / count
    var = ss / count - mean * mean
    scale = gamma * jax.lax.rsqrt(var + _EPS)
    shift = beta - mean * scale
    return scale.reshape(-1, 1), shift.reshape(-1, 1)


def kernel(x_nchw, w1, w2, g1, b1, g2, b2):
    N, C, H, W = x_nchw.shape
    HW = H * W
    x = x_nchw.reshape(N, C, HW)
    count = N * HW

    w1t = w1.reshape(9 * C, C).T.astype(jnp.bfloat16)
    w2t = w2.reshape(9 * C, C).T.astype(jnp.bfloat16)

    c1, st1 = _conv_bnstats(x, w1t, W)
    sc1, sh1 = _fold(st1, count, g1, b1)
    c2, st2 = _conv_bnstats(c1, w2t, W, sc1, sh1)
    sc2, sh2 = _fold(st2, count, g2, b2)

    out = pl.pallas_call(
        _tail_kernel,
        grid=(N,),
        in_specs=[
            pl.BlockSpec((1, C, HW), lambda n: (n, 0, 0)),
            pl.BlockSpec((1, C, HW), lambda n: (n, 0, 0)),
            pl.BlockSpec((C, 1), lambda n: (0, 0)),
            pl.BlockSpec((C, 1), lambda n: (0, 0)),
        ],
        out_specs=pl.BlockSpec((1, C, HW), lambda n: (n, 0, 0)),
        out_shape=jax.ShapeDtypeStruct((N, C, HW), jnp.float32),
        compiler_params=_cparams(("parallel",)),
    )(c2, x, sc2, sh2)
    return out.reshape(N, C, H, W)
